# Initial kernel scaffold; baseline (speedup 1.0000x reference)
import jax
import jax.numpy as jnp
from jax.experimental import pallas as pl


def kernel(z, codebook):
    raise NotImplementedError("write your pallas kernel here")


# speedup vs baseline: 1.0280x; 1.0280x over previous
import functools

import jax
import jax.numpy as jnp
from jax import lax
from jax.experimental import pallas as pl
from jax.experimental.pallas import tpu as pltpu
from jax.experimental.pallas import tpu_sc as plsc

N_CODES = 8192
DIM = 256
TOKENS = 8192
TM = 256
WIN = 2048
BETA = 0.25

SC_CORES = 2
SC_SUBCORES = 16
SC_WORKERS = SC_CORES * SC_SUBCORES
ROWS_PER_WORKER = TOKENS // SC_WORKERS
IDX_CHUNK = 128
CHUNKS_PER_WORKER = ROWS_PER_WORKER // IDX_CHUNK


def _argmin_body(zf_ref, cb_ref, zn_ref, cbn_ref, idx_ref, loss_ref):
    t = pl.program_id(0)
    zf = zf_ref[...]
    zn = zn_ref[...]
    acc_v = jnp.full((TM, 1), jnp.inf, jnp.float32)
    acc_i = jnp.zeros((TM, 1), jnp.int32)
    wins = []
    for w in range(N_CODES // WIN):
        cb_w = cb_ref[w * WIN:(w + 1) * WIN, :]
        mm = lax.dot_general(zf, cb_w, (((1,), (1,)), ((), ())),
                             preferred_element_type=jnp.float32)
        cbn_w = cbn_ref[:, w * WIN:(w + 1) * WIN]
        score = (zn + cbn_w) - 2.0 * mm
        wv = jnp.min(score, axis=1, keepdims=True)
        ids = lax.broadcasted_iota(jnp.int32, score.shape, 1) + w * WIN
        wi = jnp.min(jnp.where(score == wv, ids, jnp.int32(2**31 - 1)),
                     axis=1, keepdims=True)
        wins.append((wv, wi))
        keep = (acc_v < wv) | ((acc_v == wv) & (acc_i < wi))
        acc_v = jnp.where(keep, acc_v, wv).astype(jnp.bfloat16).astype(jnp.float32)
        acc_i = jnp.where(keep, acc_i, wi)
    idx_ref[...] = acc_i

    d_sel = jnp.full((TM, 1), jnp.inf, jnp.float32)
    for wv, wi in wins:
        d_sel = jnp.where(wi == acc_i, wv, d_sel)

    @pl.when(t == 0)
    def _():
        loss_ref[0, 0] = 0.0

    loss_ref[0, 0] += jnp.sum(d_sel)


_ARGMIN_CALL = pl.pallas_call(
    _argmin_body,
    grid=(TOKENS // TM,),
    in_specs=[
        pl.BlockSpec((TM, DIM), lambda t: (t, 0)),
        pl.BlockSpec((N_CODES, DIM), lambda t: (0, 0)),
        pl.BlockSpec((TM, 1), lambda t: (t, 0)),
        pl.BlockSpec((1, N_CODES), lambda t: (0, 0)),
    ],
    out_specs=[
        pl.BlockSpec((TM, 1), lambda t: (t, 0)),
        pl.BlockSpec(memory_space=pltpu.SMEM),
    ],
    out_shape=[
        jax.ShapeDtypeStruct((TOKENS, 1), jnp.int32),
        jax.ShapeDtypeStruct((1, 1), jnp.float32),
    ],
)


def _gather_body(cb_hbm, idx_hbm, out_hbm, idx_v, rows_v, sem):
    wid = lax.axis_index("s") * SC_CORES + lax.axis_index("c")
    pltpu.sync_copy(idx_hbm.at[pl.ds(wid * CHUNKS_PER_WORKER,
                                     CHUNKS_PER_WORKER)], idx_v)
    copies = []
    for j in range(CHUNKS_PER_WORKER):
        copies.append(pltpu.async_copy(
            cb_hbm.at[idx_v.at[j]],
            rows_v.at[pl.ds(j * IDX_CHUNK, IDX_CHUNK)], sem))
    for cp in copies:
        cp.wait()
    pltpu.sync_copy(rows_v, out_hbm.at[pl.ds(wid * ROWS_PER_WORKER,
                                             ROWS_PER_WORKER)])


@functools.cache
def _gather_call():
    return functools.partial(
        pl.kernel,
        mesh=plsc.VectorSubcoreMesh(core_axis_name="c", subcore_axis_name="s"),
        out_type=jax.ShapeDtypeStruct((TOKENS, DIM), jnp.float32),
        scratch_types=[
            pltpu.VMEM((CHUNKS_PER_WORKER, IDX_CHUNK), jnp.int32),
            pltpu.VMEM((ROWS_PER_WORKER, DIM), jnp.float32),
            pltpu.SemaphoreType.DMA,
        ],
    )(_gather_body)


def kernel(z, codebook):
    zp = jnp.transpose(z, (0, 2, 3, 1))
    zf = zp.reshape(-1, DIM)
    zn = jnp.sum(zf ** 2, axis=1, keepdims=True)
    cbn = jnp.sum(codebook ** 2, axis=1).reshape(1, N_CODES)
    zf_bf = zf.astype(jnp.bfloat16)
    cb_bf = codebook.astype(jnp.bfloat16)
    idx2, loss_sum = _ARGMIN_CALL(zf_bf, cb_bf, zn, cbn)
    idx = idx2.reshape(SC_WORKERS * CHUNKS_PER_WORKER, IDX_CHUNK)
    cb_q = lax.reduce_precision(codebook, exponent_bits=8, mantissa_bits=7)
    zq_flat = _gather_call()(cb_q, idx)
    zq = zq_flat.reshape(z.shape[0], z.shape[2], z.shape[3], DIM)
    zq = jnp.transpose(zq, (0, 3, 1, 2))
    z_q_st = z + (zq - z)
    m = loss_sum[0, 0] / jnp.float32(z.size)
    loss = m + BETA * m
    return (loss, z_q_st)


# speedup vs baseline: 1.1165x; 1.0861x over previous
import functools

import jax
import jax.numpy as jnp
from jax import lax
from jax.experimental import pallas as pl
from jax.experimental.pallas import tpu as pltpu
from jax.experimental.pallas import tpu_sc as plsc

N_CODES = 8192
DIM = 256
TOKENS = 8192
TM = 512
WIN = 2048
BETA = 0.25

SC_CORES = 2
SC_SUBCORES = 16
SC_WORKERS = SC_CORES * SC_SUBCORES
ROWS_PER_WORKER = TOKENS // SC_WORKERS
IDX_CHUNK = 128
CHUNKS_PER_WORKER = ROWS_PER_WORKER // IDX_CHUNK


def _argmin_body(zf_ref, cb_ref, zn_ref, cbn_ref, idx_ref, loss_ref):
    t = pl.program_id(0)
    zf = zf_ref[...]
    zn = zn_ref[...]
    acc_v = jnp.full((TM, 1), jnp.inf, jnp.float32)
    acc_i = jnp.zeros((TM, 1), jnp.int32)
    wins = []
    for w in range(N_CODES // WIN):
        cb_w = cb_ref[w * WIN:(w + 1) * WIN, :]
        mm = lax.dot_general(zf, cb_w, (((1,), (1,)), ((), ())),
                             preferred_element_type=jnp.float32)
        cbn_w = cbn_ref[:, w * WIN:(w + 1) * WIN]
        score = (zn + cbn_w) + mm
        wv = jnp.min(score, axis=1, keepdims=True)
        ids = lax.broadcasted_iota(jnp.int32, score.shape, 1) + w * WIN
        wi = jnp.min(jnp.where(score == wv, ids, jnp.int32(2**31 - 1)),
                     axis=1, keepdims=True)
        wins.append((wv, wi))
        keep = (acc_v < wv) | ((acc_v == wv) & (acc_i < wi))
        acc_v = jnp.where(keep, acc_v, wv).astype(jnp.bfloat16).astype(jnp.float32)
        acc_i = jnp.where(keep, acc_i, wi)
    idx_ref[...] = acc_i

    d_sel = jnp.full((TM, 1), jnp.inf, jnp.float32)
    for wv, wi in wins:
        d_sel = jnp.where(wi == acc_i, wv, d_sel)

    @pl.when(t == 0)
    def _():
        loss_ref[0, 0] = 0.0

    loss_ref[0, 0] += jnp.sum(d_sel)


_ARGMIN_CALL = pl.pallas_call(
    _argmin_body,
    grid=(TOKENS // TM,),
    in_specs=[
        pl.BlockSpec((TM, DIM), lambda t: (t, 0)),
        pl.BlockSpec((N_CODES, DIM), lambda t: (0, 0)),
        pl.BlockSpec((TM, 1), lambda t: (t, 0)),
        pl.BlockSpec((1, N_CODES), lambda t: (0, 0)),
    ],
    out_specs=[
        pl.BlockSpec((TM, 1), lambda t: (t, 0)),
        pl.BlockSpec(memory_space=pltpu.SMEM),
    ],
    out_shape=[
        jax.ShapeDtypeStruct((TOKENS, 1), jnp.int32),
        jax.ShapeDtypeStruct((1, 1), jnp.float32),
    ],
)


def _gather_body(cb_hbm, idx_hbm, out_hbm, idx_v, rows_v, sem):
    wid = lax.axis_index("s") * SC_CORES + lax.axis_index("c")
    pltpu.sync_copy(idx_hbm.at[pl.ds(wid * CHUNKS_PER_WORKER,
                                     CHUNKS_PER_WORKER)], idx_v)
    copies = []
    for j in range(CHUNKS_PER_WORKER):
        copies.append(pltpu.async_copy(
            cb_hbm.at[idx_v.at[j]],
            rows_v.at[pl.ds(j * IDX_CHUNK, IDX_CHUNK)], sem))
    for cp in copies:
        cp.wait()
    pltpu.sync_copy(rows_v, out_hbm.at[pl.ds(wid * ROWS_PER_WORKER,
                                             ROWS_PER_WORKER)])


@functools.cache
def _gather_call():
    return functools.partial(
        pl.kernel,
        mesh=plsc.VectorSubcoreMesh(core_axis_name="c", subcore_axis_name="s"),
        out_type=jax.ShapeDtypeStruct((TOKENS, DIM), jnp.float32),
        scratch_types=[
            pltpu.VMEM((CHUNKS_PER_WORKER, IDX_CHUNK), jnp.int32),
            pltpu.VMEM((ROWS_PER_WORKER, DIM), jnp.float32),
            pltpu.SemaphoreType.DMA,
        ],
    )(_gather_body)


def kernel(z, codebook):
    zp = jnp.transpose(z, (0, 2, 3, 1))
    zf = zp.reshape(-1, DIM)
    zn = jnp.sum(zf ** 2, axis=1, keepdims=True)
    cbn = jnp.sum(codebook ** 2, axis=1).reshape(1, N_CODES)
    zf_bf = zf.astype(jnp.bfloat16) * jnp.bfloat16(-2.0)
    cb_bf = codebook.astype(jnp.bfloat16)
    idx2, loss_sum = _ARGMIN_CALL(zf_bf, cb_bf, zn, cbn)
    idx = idx2.reshape(SC_WORKERS * CHUNKS_PER_WORKER, IDX_CHUNK)
    cb_q = lax.reduce_precision(codebook, exponent_bits=8, mantissa_bits=7)
    zq_flat = _gather_call()(cb_q, idx)
    zq = zq_flat.reshape(z.shape[0], z.shape[2], z.shape[3], DIM)
    zq = jnp.transpose(zq, (0, 3, 1, 2))
    z_q_st = z + (zq - z)
    m = loss_sum[0, 0] / jnp.float32(z.size)
    loss = m + BETA * m
    return (loss, z_q_st)


# speedup vs baseline: 1.1678x; 1.0460x over previous
import functools

import jax
import jax.numpy as jnp
from jax import lax
from jax.experimental import pallas as pl
from jax.experimental.pallas import tpu as pltpu
from jax.experimental.pallas import tpu_sc as plsc

N_CODES = 8192
DIM = 256
TOKENS = 8192
TM = 1024
WIN = 2048
BETA = 0.25

SC_CORES = 2
SC_SUBCORES = 16
SC_WORKERS = SC_CORES * SC_SUBCORES
ROWS_PER_WORKER = TOKENS // SC_WORKERS
IDX_CHUNK = 128
CHUNKS_PER_WORKER = ROWS_PER_WORKER // IDX_CHUNK


def _argmin_body(zf_ref, cb_ref, zn_ref, cbn_ref, idx_ref, loss_ref):
    t = pl.program_id(0)
    zf = zf_ref[...]
    zn = zn_ref[...]
    acc_v = jnp.full((TM, 1), jnp.inf, jnp.float32)
    acc_i = jnp.zeros((TM, 1), jnp.int32)
    wins = []
    for w in range(N_CODES // WIN):
        cb_w = cb_ref[w * WIN:(w + 1) * WIN, :]
        mm = lax.dot_general(zf, cb_w, (((1,), (1,)), ((), ())),
                             preferred_element_type=jnp.float32)
        cbn_w = cbn_ref[:, w * WIN:(w + 1) * WIN]
        score = (zn + cbn_w) + mm
        wv = jnp.min(score, axis=1, keepdims=True)
        ids = lax.broadcasted_iota(jnp.int32, score.shape, 1) + w * WIN
        wi = jnp.min(jnp.where(score == wv, ids, jnp.int32(2**31 - 1)),
                     axis=1, keepdims=True)
        wins.append((wv, wi))
        keep = (acc_v < wv) | ((acc_v == wv) & (acc_i < wi))
        acc_v = jnp.where(keep, acc_v, wv).astype(jnp.bfloat16).astype(jnp.float32)
        acc_i = jnp.where(keep, acc_i, wi)
    idx_ref[...] = acc_i

    d_sel = jnp.full((TM, 1), jnp.inf, jnp.float32)
    for wv, wi in wins:
        d_sel = jnp.where(wi == acc_i, wv, d_sel)

    @pl.when(t == 0)
    def _():
        loss_ref[0, 0] = 0.0

    loss_ref[0, 0] += jnp.sum(d_sel)


_ARGMIN_CALL = pl.pallas_call(
    _argmin_body,
    grid=(TOKENS // TM,),
    in_specs=[
        pl.BlockSpec((TM, DIM), lambda t: (t, 0)),
        pl.BlockSpec((N_CODES, DIM), lambda t: (0, 0)),
        pl.BlockSpec((TM, 1), lambda t: (t, 0)),
        pl.BlockSpec((1, N_CODES), lambda t: (0, 0)),
    ],
    out_specs=[
        pl.BlockSpec((TM, 1), lambda t: (t, 0)),
        pl.BlockSpec(memory_space=pltpu.SMEM),
    ],
    out_shape=[
        jax.ShapeDtypeStruct((TOKENS, 1), jnp.int32),
        jax.ShapeDtypeStruct((1, 1), jnp.float32),
    ],
)


def _gather_body(cb_hbm, idx_hbm, out_hbm, idx_v, rows_v, sem):
    wid = lax.axis_index("s") * SC_CORES + lax.axis_index("c")
    pltpu.sync_copy(idx_hbm.at[pl.ds(wid * CHUNKS_PER_WORKER,
                                     CHUNKS_PER_WORKER)], idx_v)
    copies = []
    for j in range(CHUNKS_PER_WORKER):
        copies.append(pltpu.async_copy(
            cb_hbm.at[idx_v.at[j]],
            rows_v.at[pl.ds(j * IDX_CHUNK, IDX_CHUNK)], sem))
    for cp in copies:
        cp.wait()
    pltpu.sync_copy(rows_v, out_hbm.at[pl.ds(wid * ROWS_PER_WORKER,
                                             ROWS_PER_WORKER)])


@functools.cache
def _gather_call():
    return functools.partial(
        pl.kernel,
        mesh=plsc.VectorSubcoreMesh(core_axis_name="c", subcore_axis_name="s"),
        out_type=jax.ShapeDtypeStruct((TOKENS, DIM), jnp.float32),
        scratch_types=[
            pltpu.VMEM((CHUNKS_PER_WORKER, IDX_CHUNK), jnp.int32),
            pltpu.VMEM((ROWS_PER_WORKER, DIM), jnp.float32),
            pltpu.SemaphoreType.DMA,
        ],
    )(_gather_body)


def kernel(z, codebook):
    zp = jnp.transpose(z, (0, 2, 3, 1))
    zf = zp.reshape(-1, DIM)
    zn = jnp.sum(zf ** 2, axis=1, keepdims=True)
    cbn = jnp.sum(codebook ** 2, axis=1).reshape(1, N_CODES)
    zf_bf = zf.astype(jnp.bfloat16) * jnp.bfloat16(-2.0)
    cb_bf = codebook.astype(jnp.bfloat16)
    idx2, loss_sum = _ARGMIN_CALL(zf_bf, cb_bf, zn, cbn)
    idx = idx2.reshape(SC_WORKERS * CHUNKS_PER_WORKER, IDX_CHUNK)
    cb_q = lax.reduce_precision(codebook, exponent_bits=8, mantissa_bits=7)
    zq_flat = _gather_call()(cb_q, idx)
    zq = zq_flat.reshape(z.shape[0], z.shape[2], z.shape[3], DIM)
    zq = jnp.transpose(zq, (0, 3, 1, 2))
    z_q_st = z + (zq - z)
    m = loss_sum[0, 0] / jnp.float32(z.size)
    loss = m + BETA * m
    return (loss, z_q_st)
